# baseline (speedup 1.0000x reference)
import jax
import jax.numpy as jnp
from jax import lax
from jax.experimental import pallas as pl
from jax.experimental.pallas import tpu as pltpu
from jax.experimental.pallas import tpu_sc as plsc

N_NODES = 10000
N_HEDGES = 10000
NNZ = 320000
D = 128
HROWS = 80

NC = 2
NS = 16
NW = NC * NS
CH = 80
PER_TILE = NNZ // NW
N_CH = PER_TILE // CH
WB_TILES = 10
WB_ROWS = 10000 // WB_TILES

_MESH = plsc.VectorSubcoreMesh(core_axis_name="c", subcore_axis_name="s")
_SC_PARAMS = pltpu.CompilerParams(needs_layout_passes=False)


def _deg_body(nidx_h, eidx_h, w_h, bh_o, dh_o,
              nidx_t, eidx_t, w_v, hist_b, hist_d, rows):
    cid = lax.axis_index("c")
    sid = lax.axis_index("s")
    wid = cid * NS + sid
    ones16 = jnp.ones((16,), jnp.float32)

    pltpu.sync_copy(nidx_h.at[wid], nidx_t)
    pltpu.sync_copy(eidx_h.at[wid], eidx_t)
    pltpu.sync_copy(w_h, w_v)

    def zrow(i, carry):
        hist_b[pl.ds(i * 16, 16)] = jnp.zeros((16,), jnp.float32)
        hist_d[pl.ds(i * 16, 16)] = jnp.zeros((16,), jnp.float32)
        return carry
    lax.fori_loop(0, HROWS * D // 16, zrow, 0)

    def step(i, carry):
        for j in range(CH // 16):
            nj = nidx_t[0, pl.ds(i * CH + j * 16, 16)]
            ej = eidx_t[0, pl.ds(i * CH + j * 16, 16)]
            plsc.addupdate_scatter(hist_b, [ej], ones16)
            vals = plsc.load_gather(w_v, [ej])
            plsc.addupdate_scatter(hist_d, [nj], vals)
        return carry
    lax.fori_loop(0, N_CH, step, 0)

    for hist, out in ((hist_b, bh_o), (hist_d, dh_o)):
        def hcp(i, carry, _h=hist):
            for j in range(D // 16):
                rows[i, pl.ds(j * 16, 16)] = _h[pl.ds(i * D + j * 16, 16)]
            return carry
        lax.fori_loop(0, HROWS, hcp, 0)
        pltpu.sync_copy(rows, out.at[wid])


_deg = pl.kernel(
    _deg_body, mesh=_MESH,
    out_type=[jax.ShapeDtypeStruct((NW, HROWS, D), jnp.float32),
              jax.ShapeDtypeStruct((NW, HROWS, D), jnp.float32)],
    scratch_types=[pltpu.VMEM((1, PER_TILE), jnp.int32),
                   pltpu.VMEM((1, PER_TILE), jnp.int32),
                   pltpu.VMEM((10000,), jnp.float32),
                   pltpu.VMEM((HROWS * D,), jnp.float32),
                   pltpu.VMEM((HROWS * D,), jnp.float32),
                   pltpu.VMEM((HROWS, D), jnp.float32)],
    compiler_params=_SC_PARAMS)


def _agg_body(table, sidx_h, didx_h, zeros_h, part_o,
              didx_t, rows_a, rows_b, sem_a, sem_b, sca, scb,
              sbufs, ssems, acc):
    cid = lax.axis_index("c")
    sid = lax.axis_index("s")
    wid = cid * NS + sid

    pltpu.sync_copy(didx_h.at[wid], didx_t)

    @pl.when(sid < WB_TILES)
    def _():
        pltpu.sync_copy(zeros_h, acc.at[pl.ds(sid * WB_ROWS, WB_ROWS)])

    plsc.subcore_barrier()

    def sload(i, k):
        pltpu.async_copy(sidx_h.at[wid, i], sbufs[k], ssems[k])

    def swait(i, k):
        pltpu.make_async_copy(sidx_h.at[wid, i], sbufs[k], ssems[k]).wait()

    def gather(rows, sem, k):
        pltpu.async_copy(table.at[sbufs[k].at[0]], rows, sem)

    def gwait(rows, sem, k):
        pltpu.make_async_copy(table.at[sbufs[k].at[0]], rows, sem).wait()

    sload(0, 0)
    swait(0, 0)
    gather(rows_a, sem_a, 0)
    sload(1, 1)
    swait(1, 1)
    gather(rows_b, sem_b, 1)

    def pair(g, carry):
        i0 = 2 * g
        gwait(rows_a, sem_a, 0)
        pltpu.async_copy(rows_a, acc.at[didx_t.at[i0]], sca, add=True)
        sload(i0 + 2, 0)
        gwait(rows_b, sem_b, 1)
        pltpu.async_copy(rows_b, acc.at[didx_t.at[i0 + 1]], scb, add=True)

        @pl.when(i0 + 3 < N_CH)
        def _():
            sload(i0 + 3, 1)
        pltpu.make_async_copy(rows_a, acc.at[didx_t.at[i0]], sca).wait()
        swait(i0 + 2, 0)
        gather(rows_a, sem_a, 0)
        pltpu.make_async_copy(rows_b, acc.at[didx_t.at[i0 + 1]], scb).wait()

        @pl.when(i0 + 3 < N_CH)
        def _():
            swait(i0 + 3, 1)
            gather(rows_b, sem_b, 1)
        return carry
    lax.fori_loop(0, (N_CH - 1) // 2, pair, 0)
    gwait(rows_a, sem_a, 0)
    pltpu.sync_copy(rows_a, acc.at[didx_t.at[N_CH - 1]], add=True)

    plsc.subcore_barrier()

    @pl.when(sid < WB_TILES)
    def _():
        r0 = sid * WB_ROWS
        pltpu.sync_copy(acc.at[pl.ds(r0, WB_ROWS)],
                        part_o.at[cid].at[pl.ds(r0, WB_ROWS)])


_agg = pl.kernel(
    _agg_body, mesh=_MESH,
    out_type=[jax.ShapeDtypeStruct((NC, 10000, D), jnp.float32)],
    scratch_types=[pltpu.VMEM((N_CH, CH), jnp.int32),
                   pltpu.VMEM((CH, D), jnp.float32),
                   pltpu.VMEM((CH, D), jnp.float32),
                   pltpu.SemaphoreType.DMA,
                   pltpu.SemaphoreType.DMA,
                   pltpu.SemaphoreType.DMA,
                   pltpu.SemaphoreType.DMA,
                   [pltpu.VMEM((1, CH), jnp.int32) for _ in range(2)],
                   [pltpu.SemaphoreType.DMA for _ in range(2)],
                   pltpu.VMEM_SHARED((10000, D), jnp.float32)],
    compiler_params=_SC_PARAMS)


def _mm_body(x_ref, w_ref, o_ref):
    o_ref[...] = lax.dot_general(x_ref[...], w_ref[...],
                                 (((1,), (1,)), ((), ())),
                                 preferred_element_type=jnp.float32)


def _mm(x, w):
    return pl.pallas_call(
        _mm_body,
        out_shape=jax.ShapeDtypeStruct((x.shape[0], w.shape[0]), jnp.float32),
    )(x, w)


def _safe_recip(d):
    return jnp.where(d > 0, 1.0 / jnp.where(d > 0, d, 1.0), 0.0)


def _degred_body(hp_ref, o_ref):
    o_ref[...] = _safe_recip(jnp.sum(hp_ref[...], axis=0))


def _degred(hp):
    return pl.pallas_call(
        _degred_body,
        out_shape=jax.ShapeDtypeStruct((HROWS, D), jnp.float32),
    )(hp)


def _scale_body(pe_ref, binv_ref, oe_ref):
    pe = pe_ref[...]
    oe_ref[...] = (pe[0] + pe[1]) * binv_ref[...]


def _scale(pe, binv):
    return pl.pallas_call(
        _scale_body,
        out_shape=jax.ShapeDtypeStruct((N_HEDGES, D), jnp.float32),
    )(pe, binv)


def _mid_body(pn_ref, dinv_ref, b1_ref, g_ref, be_ref, w2_ref, o_ref):
    pn = pn_ref[...]
    h = (pn[0] + pn[1]) * dinv_ref[...] + b1_ref[...][None, :]
    mu = jnp.mean(h, axis=0, keepdims=True)
    var = jnp.mean((h - mu) ** 2, axis=0, keepdims=True)
    hn = (g_ref[...][None, :] * (h - mu) / jnp.sqrt(var + 1e-5)
          + be_ref[...][None, :])
    o_ref[...] = lax.dot_general(hn, w2_ref[...],
                                 (((1,), (1,)), ((), ())),
                                 preferred_element_type=jnp.float32)


def _mid(pn, dinv, b1, gamma, beta, W2):
    return pl.pallas_call(
        _mid_body,
        out_shape=jax.ShapeDtypeStruct((N_NODES, D), jnp.float32),
    )(pn, dinv, b1, gamma, beta, W2)


def _final_body(pn_ref, dinv_ref, b2_ref, o_ref):
    pn = pn_ref[...]
    o_ref[...] = (pn[0] + pn[1]) * dinv_ref[...] + b2_ref[...][None, :]


def _final(pn, dinv, b2):
    return pl.pallas_call(
        _final_body,
        out_shape=jax.ShapeDtypeStruct((N_NODES, D), jnp.float32),
    )(pn, dinv, b2)


def kernel(x, hyperedge_index, hyperedge_weight, W1, b1, gamma, beta, W2, b2):
    nidx_f = hyperedge_index[0].reshape(NW, 1, PER_TILE)
    eidx_f = hyperedge_index[1].reshape(NW, 1, PER_TILE)
    nidx4 = hyperedge_index[0].reshape(NW, N_CH, 1, CH)
    eidx4 = hyperedge_index[1].reshape(NW, N_CH, 1, CH)
    nidx3 = hyperedge_index[0].reshape(NW, N_CH, CH)
    eidx3 = hyperedge_index[1].reshape(NW, N_CH, CH)
    zeros_h = jnp.zeros((WB_ROWS, D), jnp.float32)

    bhist, dhist = _deg(nidx_f, eidx_f, hyperedge_weight)
    binv = _degred(bhist).reshape(HROWS * D, 1)[:N_HEDGES]
    dinv = _degred(dhist).reshape(HROWS * D, 1)[:N_NODES]

    xw1 = _mm(x, W1)
    (pe1,) = _agg(xw1, nidx4, eidx3, zeros_h)
    oe1 = _scale(pe1, binv)
    (pn1,) = _agg(oe1, eidx4, nidx3, zeros_h)
    xw2 = _mid(pn1, dinv, b1, gamma, beta, W2)
    (pe2,) = _agg(xw2, nidx4, eidx3, zeros_h)
    oe2 = _scale(pe2, binv)
    (pn2,) = _agg(oe2, eidx4, nidx3, zeros_h)
    return _final(pn2, dinv, b2)


# speedup vs baseline: 1.2031x; 1.2031x over previous
import jax
import jax.numpy as jnp
from jax import lax
from jax.experimental import pallas as pl
from jax.experimental.pallas import tpu as pltpu
from jax.experimental.pallas import tpu_sc as plsc

N_NODES = 10000
N_HEDGES = 10000
NNZ = 320000
D = 128
HROWS = 80

NC = 2
NS = 16
NW = NC * NS
CH = 80
PER_TILE = NNZ // NW
N_CH = PER_TILE // CH
WB_TILES = 10
WB_ROWS = 10000 // WB_TILES

_MESH = plsc.VectorSubcoreMesh(core_axis_name="c", subcore_axis_name="s")
_SC_PARAMS = pltpu.CompilerParams(needs_layout_passes=False)


def _deg_body(nidx_h, eidx_h, w_h, bh_o, dh_o,
              nidx_t, eidx_t, w_v, hist_b, hist_d, rows):
    cid = lax.axis_index("c")
    sid = lax.axis_index("s")
    wid = cid * NS + sid
    ones16 = jnp.ones((16,), jnp.float32)

    pltpu.sync_copy(nidx_h.at[wid], nidx_t)
    pltpu.sync_copy(eidx_h.at[wid], eidx_t)
    pltpu.sync_copy(w_h, w_v)

    def zrow(i, carry):
        hist_b[pl.ds(i * 16, 16)] = jnp.zeros((16,), jnp.float32)
        hist_d[pl.ds(i * 16, 16)] = jnp.zeros((16,), jnp.float32)
        return carry
    lax.fori_loop(0, HROWS * D // 16, zrow, 0)

    def step(i, carry):
        for j in range(CH // 16):
            nj = nidx_t[0, pl.ds(i * CH + j * 16, 16)]
            ej = eidx_t[0, pl.ds(i * CH + j * 16, 16)]
            plsc.addupdate_scatter(hist_b, [ej], ones16)
            vals = plsc.load_gather(w_v, [ej])
            plsc.addupdate_scatter(hist_d, [nj], vals)
        return carry
    lax.fori_loop(0, N_CH, step, 0)

    for hist, out in ((hist_b, bh_o), (hist_d, dh_o)):
        def hcp(i, carry, _h=hist):
            for j in range(D // 16):
                rows[i, pl.ds(j * 16, 16)] = _h[pl.ds(i * D + j * 16, 16)]
            return carry
        lax.fori_loop(0, HROWS, hcp, 0)
        pltpu.sync_copy(rows, out.at[wid])


_deg = pl.kernel(
    _deg_body, mesh=_MESH,
    out_type=[jax.ShapeDtypeStruct((NW, HROWS, D), jnp.float32),
              jax.ShapeDtypeStruct((NW, HROWS, D), jnp.float32)],
    scratch_types=[pltpu.VMEM((1, PER_TILE), jnp.int32),
                   pltpu.VMEM((1, PER_TILE), jnp.int32),
                   pltpu.VMEM((10000,), jnp.float32),
                   pltpu.VMEM((HROWS * D,), jnp.float32),
                   pltpu.VMEM((HROWS * D,), jnp.float32),
                   pltpu.VMEM((HROWS, D), jnp.float32)],
    compiler_params=_SC_PARAMS)


def _agg_body(table, sidx_h, didx_h, zeros_h, part_o,
              didx_t, rows, gsems, csems, sbufs, ssems, acc):
    cid = lax.axis_index("c")
    sid = lax.axis_index("s")
    wid = cid * NS + sid

    pltpu.sync_copy(didx_h.at[wid], didx_t)

    @pl.when(sid < WB_TILES)
    def _():
        pltpu.sync_copy(zeros_h, acc.at[pl.ds(sid * WB_ROWS, WB_ROWS)])

    plsc.subcore_barrier()

    nbuf = len(rows)

    def sload(i, k):
        pltpu.async_copy(sidx_h.at[wid, i], sbufs[k], ssems[k])

    def swait(i, k):
        pltpu.make_async_copy(sidx_h.at[wid, i], sbufs[k], ssems[k]).wait()

    def gather(k):
        pltpu.async_copy(table.at[sbufs[k].at[0]], rows[k], gsems[k])

    def gwait(k):
        pltpu.make_async_copy(table.at[sbufs[k].at[0]], rows[k],
                              gsems[k]).wait()

    def scat(i, k):
        pltpu.async_copy(rows[k], acc.at[didx_t.at[i]], csems[k], add=True)

    def scwait(i, k):
        pltpu.make_async_copy(rows[k], acc.at[didx_t.at[i]],
                              csems[k]).wait()

    for k in range(nbuf):
        sload(k, k)
        swait(k, k)
        gather(k)

    def trio(g, carry):
        c = nbuf * g
        for k in range(nbuf):
            gwait(k)
            scat(c + k, k)
            sload(c + k + nbuf, k)
        for k in range(nbuf):
            scwait(c + k, k)
            swait(c + k + nbuf, k)
            gather(k)
        return carry
    lax.fori_loop(0, N_CH // nbuf - 1, trio, 0)
    c0 = (N_CH // nbuf - 1) * nbuf
    for k in range(nbuf):
        gwait(k)
        scat(c0 + k, k)
        if c0 + k + nbuf < N_CH:
            sload(c0 + k + nbuf, k)
    for k in range(nbuf):
        scwait(c0 + k, k)
        if c0 + k + nbuf < N_CH:
            swait(c0 + k + nbuf, k)
            gather(k)
    for k in range(N_CH - c0 - nbuf):
        gwait(k)
        pltpu.sync_copy(rows[k], acc.at[didx_t.at[c0 + nbuf + k]], add=True)

    plsc.subcore_barrier()

    @pl.when(sid < WB_TILES)
    def _():
        r0 = sid * WB_ROWS
        pltpu.sync_copy(acc.at[pl.ds(r0, WB_ROWS)],
                        part_o.at[cid].at[pl.ds(r0, WB_ROWS)])


_agg = pl.kernel(
    _agg_body, mesh=_MESH,
    out_type=[jax.ShapeDtypeStruct((NC, 10000, D), jnp.float32)],
    scratch_types=[pltpu.VMEM((N_CH, CH), jnp.int32),
                   [pltpu.VMEM((CH, D), jnp.float32) for _ in range(3)],
                   [pltpu.SemaphoreType.DMA for _ in range(3)],
                   [pltpu.SemaphoreType.DMA for _ in range(3)],
                   [pltpu.VMEM((1, CH), jnp.int32) for _ in range(3)],
                   [pltpu.SemaphoreType.DMA for _ in range(3)],
                   pltpu.VMEM_SHARED((10000, D), jnp.float32)],
    compiler_params=_SC_PARAMS)


def _mm_body(x_ref, w_ref, o_ref):
    o_ref[...] = lax.dot_general(x_ref[...], w_ref[...],
                                 (((1,), (1,)), ((), ())),
                                 preferred_element_type=jnp.float32)


def _mm(x, w):
    return pl.pallas_call(
        _mm_body,
        out_shape=jax.ShapeDtypeStruct((x.shape[0], w.shape[0]), jnp.float32),
    )(x, w)


def _safe_recip(d):
    return jnp.where(d > 0, 1.0 / jnp.where(d > 0, d, 1.0), 0.0)


def _degred_body(hp_ref, o_ref):
    o_ref[...] = _safe_recip(jnp.sum(hp_ref[...], axis=0))


def _degred(hp):
    return pl.pallas_call(
        _degred_body,
        out_shape=jax.ShapeDtypeStruct((HROWS, D), jnp.float32),
    )(hp)


def _scale_body(pe_ref, binv_ref, oe_ref):
    pe = pe_ref[...]
    oe_ref[...] = (pe[0] + pe[1]) * binv_ref[...]


def _scale(pe, binv):
    return pl.pallas_call(
        _scale_body,
        out_shape=jax.ShapeDtypeStruct((N_HEDGES, D), jnp.float32),
    )(pe, binv)


def _mid_body(pn_ref, dinv_ref, b1_ref, g_ref, be_ref, w2_ref, o_ref):
    pn = pn_ref[...]
    h = (pn[0] + pn[1]) * dinv_ref[...] + b1_ref[...][None, :]
    mu = jnp.mean(h, axis=0, keepdims=True)
    var = jnp.mean((h - mu) ** 2, axis=0, keepdims=True)
    hn = (g_ref[...][None, :] * (h - mu) / jnp.sqrt(var + 1e-5)
          + be_ref[...][None, :])
    o_ref[...] = lax.dot_general(hn, w2_ref[...],
                                 (((1,), (1,)), ((), ())),
                                 preferred_element_type=jnp.float32)


def _mid(pn, dinv, b1, gamma, beta, W2):
    return pl.pallas_call(
        _mid_body,
        out_shape=jax.ShapeDtypeStruct((N_NODES, D), jnp.float32),
    )(pn, dinv, b1, gamma, beta, W2)


def _final_body(pn_ref, dinv_ref, b2_ref, o_ref):
    pn = pn_ref[...]
    o_ref[...] = (pn[0] + pn[1]) * dinv_ref[...] + b2_ref[...][None, :]


def _final(pn, dinv, b2):
    return pl.pallas_call(
        _final_body,
        out_shape=jax.ShapeDtypeStruct((N_NODES, D), jnp.float32),
    )(pn, dinv, b2)


def kernel(x, hyperedge_index, hyperedge_weight, W1, b1, gamma, beta, W2, b2):
    nidx_f = hyperedge_index[0].reshape(NW, 1, PER_TILE)
    eidx_f = hyperedge_index[1].reshape(NW, 1, PER_TILE)
    nidx4 = hyperedge_index[0].reshape(NW, N_CH, 1, CH)
    eidx4 = hyperedge_index[1].reshape(NW, N_CH, 1, CH)
    nidx3 = hyperedge_index[0].reshape(NW, N_CH, CH)
    eidx3 = hyperedge_index[1].reshape(NW, N_CH, CH)
    zeros_h = jnp.zeros((WB_ROWS, D), jnp.float32)

    bhist, dhist = _deg(nidx_f, eidx_f, hyperedge_weight)
    binv = _degred(bhist).reshape(HROWS * D, 1)[:N_HEDGES]
    dinv = _degred(dhist).reshape(HROWS * D, 1)[:N_NODES]

    xw1 = _mm(x, W1)
    (pe1,) = _agg(xw1, nidx4, eidx3, zeros_h)
    oe1 = _scale(pe1, binv)
    (pn1,) = _agg(oe1, eidx4, nidx3, zeros_h)
    xw2 = _mid(pn1, dinv, b1, gamma, beta, W2)
    (pe2,) = _agg(xw2, nidx4, eidx3, zeros_h)
    oe2 = _scale(pe2, binv)
    (pn2,) = _agg(oe2, eidx4, nidx3, zeros_h)
    return _final(pn2, dinv, b2)
